# baseline (speedup 1.0000x reference)
import functools

import jax
import jax.numpy as jnp
from jax import lax
from jax.experimental import pallas as pl
from jax.experimental.pallas import tpu as pltpu
from jax.experimental.pallas import tpu_sc as plsc

VOC = 1000000
DIM = 32
NB = 16384
NF = 26
NW = 32
NJ = VOC // 128
JPT = 245
NU = NF * (NB // 128)
UPT = NU // NW

_mesh = plsc.VectorSubcoreMesh(core_axis_name="c", subcore_axis_name="s")


@functools.partial(
    pl.kernel,
    out_type=jax.ShapeDtypeStruct((VOC // 4 + 16, 128), jnp.float32),
    mesh=_mesh,
    scratch_types=[
        pltpu.VMEM((DIM, 128), jnp.float32),
        pltpu.VMEM((32, 128), jnp.float32),
    ],
    compiler_params=pltpu.CompilerParams(use_tc_tiling_on_sc=True,
                                         needs_layout_passes=False),
)
def _detile(wt_hbm, wtail_hbm, w2_hbm, tin, tout):
  wid = lax.axis_index("s") * 2 + lax.axis_index("c")
  lo = wid * JPT
  hi = jnp.minimum(lo + JPT, NJ)
  iot = lax.iota(jnp.int32, 16)

  def block(j, carry):
    for i in range(4):
      pltpu.sync_copy(wt_hbm.at[pl.ds(8 * i, 8), pl.ds(j * 128, 128)],
                      tin.at[pl.ds(8 * i, 8)])
    for qp in range(32):
      for h in range(8):
        c_arr = iot + (16 * (h % 2))
        rr = jnp.full((16,), 4 * qp + h // 2, jnp.int32)
        tout[qp, pl.ds(16 * h, 16)] = plsc.load_gather(tin, [c_arr, rr])
    pltpu.sync_copy(tout, w2_hbm.at[pl.ds(j * 32, 32)])
    return carry

  lax.fori_loop(lo, hi, block, 0)

  @pl.when(wid == NW - 1)
  def _():
    pltpu.sync_copy(wtail_hbm, tin.at[pl.ds(0, 16)])
    pltpu.sync_copy(tin.at[pl.ds(0, 16)], w2_hbm.at[pl.ds(NJ * 32, 16)])


@functools.partial(
    pl.kernel,
    out_type=jax.ShapeDtypeStruct((NF, DIM, NB), jnp.float32),
    mesh=_mesh,
    scratch_types=[
        pltpu.VMEM((128,), jnp.int32),
        pltpu.VMEM((128,), jnp.int32),
        pltpu.VMEM((128, 128), jnp.float32),
        pltpu.VMEM((DIM, 128), jnp.float32),
        pltpu.SemaphoreType.DMA,
    ],
    compiler_params=pltpu.CompilerParams(use_tc_tiling_on_sc=True,
                                         needs_layout_passes=False),
)
def _gather_t(w2_hbm, xt_hbm, out_hbm, qidx_v, rem_v, grows, tstage, gsem):
  wid = lax.axis_index("s") * 2 + lax.axis_index("c")
  iot = lax.iota(jnp.int32, 16)

  def unit(u, carry):
    f = u // 128
    v = u % 128
    pltpu.sync_copy(xt_hbm.at[f, pl.ds(v * 128, 128)], qidx_v)
    for g in range(8):
      idx = qidx_v[pl.ds(16 * g, 16)]
      rem_v[pl.ds(16 * g, 16)] = (idx & 3) * 32
      qidx_v[pl.ds(16 * g, 16)] = idx >> 2
    pltpu.async_copy(w2_hbm.at[qidx_v], grows, gsem).wait()
    for g in range(8):
      bvec = iot + 16 * g
      col0 = rem_v[pl.ds(16 * g, 16)]
      for d in range(DIM):
        tstage[d, pl.ds(16 * g, 16)] = plsc.load_gather(grows, [bvec, col0 + d])
    for s in range(4):
      pltpu.sync_copy(tstage.at[pl.ds(8 * s, 8)],
                      out_hbm.at[f, pl.ds(8 * s, 8), pl.ds(v * 128, 128)])
    return carry

  lax.fori_loop(wid * UPT, (wid + 1) * UPT, unit, 0)


def kernel(x, weight):
  wt = weight.T
  xt = x.T
  wtail = weight[VOC - 64:].reshape(16, 128)
  w2 = _detile(wt, wtail)
  out_t = _gather_t(w2, xt)
  return out_t.transpose(2, 0, 1)


# speedup vs baseline: 1.8429x; 1.8429x over previous
import functools

import jax
import jax.numpy as jnp
from jax import lax
from jax.experimental import pallas as pl
from jax.experimental.pallas import tpu as pltpu
from jax.experimental.pallas import tpu_sc as plsc

VOC = 1000000
DIM = 32
NB = 16384
NF = 26
NW = 32
NJ = VOC // 128
JPT = 245
NU = NF * (NB // 128)
UPT = NU // NW

_mesh = plsc.VectorSubcoreMesh(core_axis_name="c", subcore_axis_name="s")


@functools.partial(
    pl.kernel,
    out_type=jax.ShapeDtypeStruct((NF, DIM, NB), jnp.float32),
    mesh=_mesh,
    scratch_types=[
        pltpu.VMEM((128,), jnp.int32),
        pltpu.VMEM((128,), jnp.int32),
        pltpu.VMEM((128, 128), jnp.float32),
        pltpu.VMEM((DIM, 128), jnp.float32),
        pltpu.SemaphoreType.DMA,
    ],
    compiler_params=pltpu.CompilerParams(use_tc_tiling_on_sc=True,
                                         needs_layout_passes=False),
)
def _gather_t(w2_hbm, xt_hbm, out_hbm, qidx_v, rem_v, grows, tstage, gsem):
  wid = lax.axis_index("s") * 2 + lax.axis_index("c")
  iot = lax.iota(jnp.int32, 16)

  def unit(u, carry):
    f = u // 128
    v = u % 128
    pltpu.sync_copy(xt_hbm.at[f, pl.ds(v * 128, 128)], qidx_v)
    for g in range(8):
      idx = qidx_v[pl.ds(16 * g, 16)]
      rem_v[pl.ds(16 * g, 16)] = (idx & 3) * 32
      qidx_v[pl.ds(16 * g, 16)] = idx >> 2
    pltpu.async_copy(w2_hbm.at[qidx_v], grows, gsem).wait()
    for g in range(8):
      bvec = iot + 16 * g
      col0 = rem_v[pl.ds(16 * g, 16)]
      for d in range(DIM):
        tstage[d, pl.ds(16 * g, 16)] = plsc.load_gather(grows, [bvec, col0 + d])
    for s in range(4):
      pltpu.sync_copy(tstage.at[pl.ds(8 * s, 8)],
                      out_hbm.at[f, pl.ds(8 * s, 8), pl.ds(v * 128, 128)])
    return carry

  lax.fori_loop(wid * UPT, (wid + 1) * UPT, unit, 0)


def kernel(x, weight):
  xt = x.T
  w2 = weight.reshape(VOC // 4, 128)
  out_t = _gather_t(w2, xt)
  return out_t.transpose(2, 0, 1)


# speedup vs baseline: 2.0814x; 1.1295x over previous
import functools

import jax
import jax.numpy as jnp
from jax import lax
from jax.experimental import pallas as pl
from jax.experimental.pallas import tpu as pltpu
from jax.experimental.pallas import tpu_sc as plsc

VOC = 1000000
DIM = 32
NB = 16384
NF = 26
NW = 32
NJ = VOC // 128
JPT = 245
NU = NF * (NB // 128)
UPT = NU // NW

_mesh = plsc.VectorSubcoreMesh(core_axis_name="c", subcore_axis_name="s")


@functools.partial(
    pl.kernel,
    out_type=jax.ShapeDtypeStruct((NF, DIM, NB), jnp.float32),
    mesh=_mesh,
    scratch_types=[
        pltpu.VMEM((128,), jnp.int32),
        pltpu.VMEM((128,), jnp.int32),
        pltpu.VMEM((128, 128), jnp.float32),
        pltpu.VMEM((DIM, 128), jnp.float32),
        pltpu.SemaphoreType.DMA,
    ],
    compiler_params=pltpu.CompilerParams(use_tc_tiling_on_sc=True,
                                         needs_layout_passes=False),
)
def _gather_t(w2_hbm, xt_hbm, out_hbm, qidx_v, rem_v, grows, tstage, gsem):
  wid = lax.axis_index("s") * 2 + lax.axis_index("c")
  iot = lax.iota(jnp.int32, 16)

  def unit(u, carry):
    f = u // 128
    v = u % 128
    pltpu.sync_copy(xt_hbm.at[f, pl.ds(v * 128, 128)], qidx_v)
    for g in range(8):
      idx = qidx_v[pl.ds(16 * g, 16)]
      rem_v[pl.ds(16 * g, 16)] = (idx & 3) * 32
      qidx_v[pl.ds(16 * g, 16)] = idx >> 2
    pltpu.async_copy(w2_hbm.at[qidx_v], grows, gsem).wait()
    for g in range(8):
      bvec = iot + 16 * g
      col0 = rem_v[pl.ds(16 * g, 16)]
      for k in range(16):
        dk = (iot + k) & 15
        for dd in range(2):
          dvec = dk + 16 * dd
          val = plsc.load_gather(grows, [bvec, col0 + dvec])
          plsc.store_scatter(tstage, [dvec, bvec], val)
    for s in range(4):
      pltpu.sync_copy(tstage.at[pl.ds(8 * s, 8)],
                      out_hbm.at[f, pl.ds(8 * s, 8), pl.ds(v * 128, 128)])
    return carry

  lax.fori_loop(wid * UPT, (wid + 1) * UPT, unit, 0)


def kernel(x, weight):
  xt = x.T
  w2 = weight.reshape(VOC // 4, 128)
  out_t = _gather_t(w2, xt)
  return out_t.transpose(2, 0, 1)


# speedup vs baseline: 2.6737x; 1.2845x over previous
import functools

import jax
import jax.numpy as jnp
from jax import lax
from jax.experimental import pallas as pl
from jax.experimental.pallas import tpu as pltpu
from jax.experimental.pallas import tpu_sc as plsc

VOC = 1000000
DIM = 32
NB = 16384
NF = 26
NW = 32
NJ = VOC // 128
JPT = 246
NU = NF * (NB // 128)
UPT = NU // NW

_mesh = plsc.VectorSubcoreMesh(core_axis_name="c", subcore_axis_name="s")


@functools.partial(
    pl.kernel,
    out_type=jax.ShapeDtypeStruct((VOC // 4 + 16, 128), jnp.float32),
    mesh=_mesh,
    scratch_types=[
        pltpu.VMEM((DIM, 128), jnp.float32),
        pltpu.VMEM((DIM, 128), jnp.float32),
        pltpu.VMEM((DIM, 128), jnp.float32),
        pltpu.VMEM((DIM, 128), jnp.float32),
        pltpu.SemaphoreType.DMA,
        pltpu.SemaphoreType.DMA,
        pltpu.SemaphoreType.DMA,
        pltpu.SemaphoreType.DMA,
    ],
    compiler_params=pltpu.CompilerParams(use_tc_tiling_on_sc=True,
                                         needs_layout_passes=False),
)
def _detile(wt_hbm, wtail_hbm, w2_hbm, tin0, tin1, tout0, tout1,
            isem0, isem1, osem0, osem1):
  wid = lax.axis_index("s") * 2 + lax.axis_index("c")
  lo = wid * JPT
  hi = jnp.minimum(lo + JPT, NJ)
  np_t = (hi - lo) // 2
  iot = lax.iota(jnp.int32, 16)
  c16 = [(iot + k) & 15 for k in range(16)]

  def issue_in(j, tin, isem):
    for i in range(4):
      pltpu.async_copy(wt_hbm.at[pl.ds(8 * i, 8), pl.ds(j * 128, 128)],
                       tin.at[pl.ds(8 * i, 8)], isem)

  def drain_in(tin, isem):
    for i in range(4):
      pltpu.make_async_copy(wt_hbm.at[pl.ds(0, 8), pl.ds(0, 128)],
                            tin.at[pl.ds(8 * i, 8)], isem).wait()

  def drain_out(tout, osem):
    pltpu.make_async_copy(tout, w2_hbm.at[pl.ds(0, 32)], osem).wait()

  def transpose(tin, tout):
    def rbody(rb, carry):
      rrv = iot + 16 * rb
      rowv = rrv >> 2
      rem32 = (rrv & 3) * 32
      for cb in range(2):
        for k in range(16):
          cvec = c16[k] + 16 * cb
          val = plsc.load_gather(tin, [cvec, rrv])
          plsc.store_scatter(tout, [rowv, rem32 + cvec], val)
      return carry

    lax.fori_loop(0, 8, rbody, 0)

  issue_in(lo, tin0, isem0)

  def pair(m, carry):
    j = lo + 2 * m
    drain_in(tin0, isem0)
    issue_in(j + 1, tin1, isem1)

    @pl.when(m > 0)
    def _():
      drain_out(tout0, osem0)
    transpose(tin0, tout0)
    pltpu.async_copy(tout0, w2_hbm.at[pl.ds(j * 32, 32)], osem0)

    drain_in(tin1, isem1)

    @pl.when(m + 1 < np_t)
    def _():
      issue_in(j + 2, tin0, isem0)

    @pl.when(m > 0)
    def _():
      drain_out(tout1, osem1)
    transpose(tin1, tout1)
    pltpu.async_copy(tout1, w2_hbm.at[pl.ds(j * 32 + 32, 32)], osem1)
    return carry

  lax.fori_loop(0, np_t, pair, 0)
  drain_out(tout0, osem0)
  drain_out(tout1, osem1)

  @pl.when(wid == NW - 1)
  def _():
    pltpu.sync_copy(wtail_hbm, tin0.at[pl.ds(0, 16)])
    pltpu.sync_copy(tin0.at[pl.ds(0, 16)], w2_hbm.at[pl.ds(NJ * 32, 16)])


@functools.partial(
    pl.kernel,
    out_type=jax.ShapeDtypeStruct((NF, DIM, NB), jnp.float32),
    mesh=_mesh,
    scratch_types=[
        pltpu.VMEM((128,), jnp.int32),
        pltpu.VMEM((128,), jnp.int32),
        pltpu.VMEM((128, 128), jnp.float32),
        pltpu.VMEM((DIM, 128), jnp.float32),
        pltpu.SemaphoreType.DMA,
    ],
    compiler_params=pltpu.CompilerParams(use_tc_tiling_on_sc=True,
                                         needs_layout_passes=False),
)
def _gather_t(w2_hbm, xt_hbm, out_hbm, qidx_v, rem_v, grows, tstage, gsem):
  wid = lax.axis_index("s") * 2 + lax.axis_index("c")
  iot = lax.iota(jnp.int32, 16)

  def unit(u, carry):
    f = u // 128
    v = u % 128
    pltpu.sync_copy(xt_hbm.at[f, pl.ds(v * 128, 128)], qidx_v)
    for g in range(8):
      idx = qidx_v[pl.ds(16 * g, 16)]
      rem_v[pl.ds(16 * g, 16)] = (idx & 3) * 32
      qidx_v[pl.ds(16 * g, 16)] = idx >> 2
    pltpu.async_copy(w2_hbm.at[qidx_v], grows, gsem).wait()
    for g in range(8):
      bvec = iot + 16 * g
      col0 = rem_v[pl.ds(16 * g, 16)]
      for k in range(16):
        dk = (iot + k) & 15
        for dd in range(2):
          dvec = dk + 16 * dd
          val = plsc.load_gather(grows, [bvec, col0 + dvec])
          plsc.store_scatter(tstage, [dvec, bvec], val)
    for s in range(4):
      pltpu.sync_copy(tstage.at[pl.ds(8 * s, 8)],
                      out_hbm.at[f, pl.ds(8 * s, 8), pl.ds(v * 128, 128)])
    return carry

  lax.fori_loop(wid * UPT, (wid + 1) * UPT, unit, 0)


def kernel(x, weight):
  wt = weight.T
  xt = x.T
  wtail = weight[VOC - 64:].reshape(16, 128)
  w2 = _detile(wt, wtail)
  out_t = _gather_t(w2, xt)
  return out_t.transpose(2, 0, 1)


# speedup vs baseline: 4.3525x; 1.6279x over previous
import functools

import jax
import jax.numpy as jnp
from jax import lax
from jax.experimental import pallas as pl
from jax.experimental.pallas import tpu as pltpu
from jax.experimental.pallas import tpu_sc as plsc

VOC = 1000000
DIM = 32
NB = 16384
NF = 26
NW = 32
NJ = VOC // 128
JPT = 246
NU = NF * (NB // 128)
UPT = NU // NW

_mesh = plsc.VectorSubcoreMesh(core_axis_name="c", subcore_axis_name="s")


@functools.partial(
    pl.kernel,
    out_type=jax.ShapeDtypeStruct((VOC // 4 + 16, 128), jnp.float32),
    mesh=_mesh,
    scratch_types=[
        pltpu.VMEM((DIM, 128), jnp.float32),
        pltpu.VMEM((DIM, 128), jnp.float32),
        pltpu.VMEM((DIM, 128), jnp.float32),
        pltpu.VMEM((DIM, 128), jnp.float32),
        pltpu.SemaphoreType.DMA,
        pltpu.SemaphoreType.DMA,
        pltpu.SemaphoreType.DMA,
        pltpu.SemaphoreType.DMA,
    ],
    compiler_params=pltpu.CompilerParams(use_tc_tiling_on_sc=True,
                                         needs_layout_passes=False),
)
def _detile(wt_hbm, wtail_hbm, w2_hbm, tin0, tin1, tout0, tout1,
            isem0, isem1, osem0, osem1):
  wid = lax.axis_index("s") * 2 + lax.axis_index("c")
  lo = wid * JPT
  hi = jnp.minimum(lo + JPT, NJ)
  np_t = (hi - lo) // 2
  iot = lax.iota(jnp.int32, 16)
  c16 = [(iot + k) & 15 for k in range(16)]

  def issue_in(j, tin, isem):
    for i in range(4):
      pltpu.async_copy(wt_hbm.at[pl.ds(8 * i, 8), pl.ds(j * 128, 128)],
                       tin.at[pl.ds(8 * i, 8)], isem)

  def drain_in(tin, isem):
    for i in range(4):
      pltpu.make_async_copy(wt_hbm.at[pl.ds(0, 8), pl.ds(0, 128)],
                            tin.at[pl.ds(8 * i, 8)], isem).wait()

  def drain_out(tout, osem):
    pltpu.make_async_copy(tout, w2_hbm.at[pl.ds(0, 32)], osem).wait()

  def transpose(tin, tout):
    def rbody(rb, carry):
      rrv = iot + 16 * rb
      rowv = rrv >> 2
      rem32 = (rrv & 3) * 32
      for cb in range(2):
        for k in range(16):
          cvec = c16[k] + 16 * cb
          val = plsc.load_gather(tin, [cvec, rrv])
          plsc.store_scatter(tout, [rowv, rem32 + cvec], val)
      return carry

    lax.fori_loop(0, 8, rbody, 0)

  issue_in(lo, tin0, isem0)

  def pair(m, carry):
    j = lo + 2 * m
    drain_in(tin0, isem0)
    issue_in(j + 1, tin1, isem1)

    @pl.when(m > 0)
    def _():
      drain_out(tout0, osem0)
    transpose(tin0, tout0)
    pltpu.async_copy(tout0, w2_hbm.at[pl.ds(j * 32, 32)], osem0)

    drain_in(tin1, isem1)

    @pl.when(m + 1 < np_t)
    def _():
      issue_in(j + 2, tin0, isem0)

    @pl.when(m > 0)
    def _():
      drain_out(tout1, osem1)
    transpose(tin1, tout1)
    pltpu.async_copy(tout1, w2_hbm.at[pl.ds(j * 32 + 32, 32)], osem1)
    return carry

  lax.fori_loop(0, np_t, pair, 0)
  drain_out(tout0, osem0)
  drain_out(tout1, osem1)

  @pl.when(wid == NW - 1)
  def _():
    pltpu.sync_copy(wtail_hbm, tin0.at[pl.ds(0, 16)])
    pltpu.sync_copy(tin0.at[pl.ds(0, 16)], w2_hbm.at[pl.ds(NJ * 32, 16)])


@functools.partial(
    pl.kernel,
    out_type=jax.ShapeDtypeStruct((NF, DIM, NB), jnp.float32),
    mesh=_mesh,
    scratch_types=[
        pltpu.VMEM((128,), jnp.int32),
        pltpu.VMEM((128,), jnp.int32),
        pltpu.VMEM((128,), jnp.int32),
        pltpu.VMEM((128,), jnp.int32),
        pltpu.VMEM((128, 128), jnp.float32),
        pltpu.VMEM((128, 128), jnp.float32),
        pltpu.VMEM((DIM, 128), jnp.float32),
        pltpu.VMEM((DIM, 128), jnp.float32),
        pltpu.SemaphoreType.DMA,
        pltpu.SemaphoreType.DMA,
        pltpu.SemaphoreType.DMA,
        pltpu.SemaphoreType.DMA,
        pltpu.SemaphoreType.DMA,
        pltpu.SemaphoreType.DMA,
    ],
    compiler_params=pltpu.CompilerParams(use_tc_tiling_on_sc=True,
                                         needs_layout_passes=False),
)
def _gather_t(w2_hbm, xt_hbm, out_hbm, q0, q1, r0, r1, g0, g1, t0, t1,
              isem0, isem1, gsem0, gsem1, osem0, osem1):
  wid = lax.axis_index("s") * 2 + lax.axis_index("c")
  u0 = wid * UPT
  iot = lax.iota(jnp.int32, 16)
  dk = [(iot + k) & 15 for k in range(16)]
  qb, rb, gb, tb = (q0, q1), (r0, r1), (g0, g1), (t0, t1)
  isem, gsem, osem = (isem0, isem1), (gsem0, gsem1), (osem0, osem1)

  def idx_copy(u, p):
    pltpu.async_copy(xt_hbm.at[u // 128, pl.ds((u % 128) * 128, 128)],
                     qb[p], isem[p])

  def drain_i(p):
    pltpu.make_async_copy(xt_hbm.at[0, pl.ds(0, 128)], qb[p], isem[p]).wait()

  def prep(p):
    for g in range(8):
      idx = qb[p][pl.ds(16 * g, 16)]
      rb[p][pl.ds(16 * g, 16)] = (idx & 3) * 32
      qb[p][pl.ds(16 * g, 16)] = idx >> 2

  def gather(p):
    pltpu.async_copy(w2_hbm.at[qb[p]], gb[p], gsem[p])

  def drain_g(p):
    pltpu.make_async_copy(w2_hbm.at[qb[p]], gb[p], gsem[p]).wait()

  def transpose(p):
    def gbody(g, carry):
      bvec = iot + 16 * g
      col0 = rb[p][pl.ds(16 * g, 16)]
      for k in range(16):
        for dd in range(2):
          dvec = dk[k] + 16 * dd
          val = plsc.load_gather(gb[p], [bvec, col0 + dvec])
          plsc.store_scatter(tb[p], [dvec, bvec], val)
      return carry

    lax.fori_loop(0, 8, gbody, 0)

  def out_copy(u, p):
    for s in range(4):
      pltpu.async_copy(
          tb[p].at[pl.ds(8 * s, 8)],
          out_hbm.at[u // 128, pl.ds(8 * s, 8), pl.ds((u % 128) * 128, 128)],
          osem[p])

  def drain_o(p):
    for s in range(4):
      pltpu.make_async_copy(tb[p].at[pl.ds(8 * s, 8)],
                            out_hbm.at[0, pl.ds(0, 8), pl.ds(0, 128)],
                            osem[p]).wait()

  idx_copy(u0, 0)
  drain_i(0)
  prep(0)
  gather(0)
  idx_copy(u0 + 1, 1)

  npair = UPT // 2

  def pair(m, carry):
    u = u0 + 2 * m
    drain_g(0)
    drain_i(1)
    prep(1)
    gather(1)

    @pl.when(m + 1 < npair)
    def _():
      idx_copy(u + 2, 0)

    @pl.when(m > 0)
    def _():
      drain_o(0)
    transpose(0)
    out_copy(u, 0)

    drain_g(1)

    @pl.when(m + 1 < npair)
    def _():
      drain_i(0)
      prep(0)
      gather(0)
      idx_copy(u + 3, 1)

    @pl.when(m > 0)
    def _():
      drain_o(1)
    transpose(1)
    out_copy(u + 1, 1)
    return carry

  lax.fori_loop(0, npair, pair, 0)
  drain_o(0)
  drain_o(1)


def kernel(x, weight):
  wt = weight.T
  xt = x.T
  wtail = weight[VOC - 64:].reshape(16, 128)
  w2 = _detile(wt, wtail)
  out_t = _gather_t(w2, xt)
  return out_t.transpose(2, 0, 1)
